# baseline (speedup 1.0000x reference)
import functools

import jax
import jax.numpy as jnp
from jax import lax
from jax.experimental import pallas as pl
from jax.experimental.pallas import tpu as pltpu
from jax.experimental.pallas import tpu_sc as plsc

D_MODEL = 128
H = 30
W = 30
NCOLORS = 11
P = H * W
B = 1024
TOTAL = B * P
NC, NS = 2, 16
NW = NC * NS
RPW = TOTAL // NW
CHUNK = 128
NCH = RPW // CHUNK


def _fused_body(color_ref, row_ref, col_ref, out_ref):
    out_ref[...] = (color_ref[...][:, None, None, :]
                    + row_ref[...][None, :, None, :]
                    + col_ref[...][None, None, :, :])


def _build_fused(color_table, row_table, col_table):
    out = pl.pallas_call(
        _fused_body,
        out_shape=jax.ShapeDtypeStruct((NCOLORS, H, W, D_MODEL), jnp.float32),
    )(color_table, row_table, col_table)
    return out.reshape(NCOLORS * P, D_MODEL)


_mesh = plsc.VectorSubcoreMesh(core_axis_name="c", subcore_axis_name="s",
                               num_cores=NC, num_subcores=NS)


NBUF = 3


@functools.partial(
    pl.kernel,
    out_type=jax.ShapeDtypeStruct((TOTAL, D_MODEL), jnp.float32),
    mesh=_mesh,
    scratch_types=[
        pltpu.VMEM((NCH, CHUNK), jnp.int32),
        pltpu.VMEM((NBUF, CHUNK, D_MODEL), jnp.float32),
        [pltpu.SemaphoreType.DMA] * NBUF,
        [pltpu.SemaphoreType.DMA] * NBUF,
    ],
)
def _sc_gather(fused_hbm, grid_hbm, out_hbm, idx_v, rows_v, gsems, ssems):
    sid = lax.axis_index("s")
    wid = sid * NC + lax.axis_index("c")
    base = wid * RPW

    pltpu.sync_copy(grid_hbm.at[wid], idx_v)

    def idx_body(j, p):
        for i in range(CHUNK // 16):
            sl = pl.ds(i * 16, 16)
            idx_v[j, sl] = idx_v[j, sl] * P + p
            p = p + 16
            p = jnp.where(p >= P, p - P, p)
        return p

    lax.fori_loop(0, NCH, idx_body, lax.iota(jnp.int32, 16))

    def g_desc(j, b):
        return pltpu.make_async_copy(
            fused_hbm.at[idx_v.at[j]], rows_v.at[b], gsems[b])

    def s_desc(j, b):
        return pltpu.make_async_copy(
            rows_v.at[b], out_hbm.at[pl.ds(base + j * CHUNK, CHUNK)], ssems[b])

    def step(j, b, wait_prev_scatter):
        g_desc(j, b).wait()
        s_desc(j, b).start()
        nb = (b + 1) % NBUF
        if wait_prev_scatter:
            s_desc(j + 1 - NBUF, nb).wait()
        return nb

    g_desc(0, 0).start()
    for j in range(NBUF):
        nb = step(j, j % NBUF, wait_prev_scatter=(j == NBUF - 1))
        g_desc(j + 1, nb).start()

    def outer(t, _):
        for b in range(NBUF):
            j = t * NBUF + b
            step(j, b, wait_prev_scatter=True)
            g_desc(j + 1, (b + 1) % NBUF).start()
        return 0

    lax.fori_loop(1, NCH // NBUF - 1, outer, 0)

    for j in range(NCH - NBUF, NCH):
        b = j % NBUF
        g_desc(j, b).wait()
        s_desc(j, b).start()
        if j + 1 < NCH:
            nb = (b + 1) % NBUF
            s_desc(j + 1 - NBUF, nb).wait()
            g_desc(j + 1, nb).start()
    for j in range(NCH - NBUF, NCH):
        s_desc(j, j % NBUF).wait()


def kernel(grid, color_table, row_table, col_table):
    fused = _build_fused(color_table, row_table, col_table)
    grid3 = grid.reshape(NW, NCH, CHUNK)
    out = _sc_gather(fused, grid3)
    return out.reshape(B, H, W, D_MODEL)


# speedup vs baseline: 1.4952x; 1.4952x over previous
import functools

import jax
import jax.numpy as jnp
from jax import lax
from jax.experimental import pallas as pl
from jax.experimental.pallas import tpu as pltpu
from jax.experimental.pallas import tpu_sc as plsc

D_MODEL = 128
H = 30
W = 30
NCOLORS = 11
P = H * W
B = 1024
NPL = B * H
NC, NS = 2, 16
NW = NC * NS
PPW = NPL // NW
PLCH = 4
NCH = PPW // PLCH
CPW = PPW * W
NBUF = 3


def _fused_body(color_ref, row_ref, col_ref, out_ref):
    out_ref[...] = (color_ref[...][:, None, None, :]
                    + row_ref[...][None, :, None, :]
                    + col_ref[...][None, None, :, :])


def _build_fused(color_table, row_table, col_table):
    out = pl.pallas_call(
        _fused_body,
        out_shape=jax.ShapeDtypeStruct((NCOLORS, H, W, D_MODEL), jnp.float32),
    )(color_table, row_table, col_table)
    return out.reshape(NCOLORS * P, D_MODEL)


_mesh = plsc.VectorSubcoreMesh(core_axis_name="c", subcore_axis_name="s",
                               num_cores=NC, num_subcores=NS)


@functools.partial(
    pl.kernel,
    out_type=jax.ShapeDtypeStruct((NPL, W, D_MODEL), jnp.float32),
    mesh=_mesh,
    compiler_params=pltpu.CompilerParams(use_tc_tiling_on_sc=True),
    scratch_types=[
        pltpu.VMEM((CPW + 16,), jnp.int32),
        pltpu.VMEM((NCH, PLCH * 32), jnp.int32),
        pltpu.VMEM((NBUF, PLCH, W, D_MODEL), jnp.float32),
        [pltpu.SemaphoreType.DMA] * NBUF,
        [pltpu.SemaphoreType.DMA] * NBUF,
    ],
)
def _sc_gather(fused_hbm, grid_hbm, out_hbm, grid_v, idx_v, rows_v,
               gsems, ssems):
    wid = lax.axis_index("s") * NC + lax.axis_index("c")
    pbase = wid * PPW

    pltpu.sync_copy(grid_hbm.at[pl.ds(wid * CPW, CPW)],
                    grid_v.at[pl.ds(0, CPW)])

    iota = lax.iota(jnp.int32, 16)

    def idx_body(c, h0):
        for k in range(PLCH):
            hk = h0 + k
            hk = jnp.where(hk >= H, hk - H, hk)
            f = c * (PLCH * W) + k * W
            pb = hk * W + iota
            idx_v[c, pl.ds(k * 32, 16)] = grid_v[pl.ds(f, 16)] * P + pb
            idx_v[c, pl.ds(k * 32 + 16, 16)] = (
                grid_v[pl.ds(f + 16, 16)] * P + pb + 16)
        h1 = h0 + PLCH
        return jnp.where(h1 >= H, h1 - H, h1)

    lax.fori_loop(0, NCH, idx_body, jnp.int32(0))

    def g_descs(c, b):
        return [pltpu.make_async_copy(
                    fused_hbm.at[idx_v.at[c, pl.ds(k * 32, W)]],
                    rows_v.at[b, k], gsems[b])
                for k in range(PLCH)]

    def s_desc(c, b):
        return pltpu.make_async_copy(
            rows_v.at[b], out_hbm.at[pl.ds(pbase + c * PLCH, PLCH)], ssems[b])

    def start_gather(c, b):
        for d in g_descs(c, b):
            d.start()

    def wait_gather(c, b):
        for d in g_descs(c, b):
            d.wait()

    def step(c, b, wait_prev_scatter):
        wait_gather(c, b)
        s_desc(c, b).start()
        nb = (b + 1) % NBUF
        if wait_prev_scatter:
            s_desc(c + 1 - NBUF, nb).wait()
        return nb

    start_gather(0, 0)
    for c in range(NBUF):
        nb = step(c, c % NBUF, wait_prev_scatter=(c == NBUF - 1))
        start_gather(c + 1, nb)

    def outer(t, _):
        for b in range(NBUF):
            c = t * NBUF + b
            step(c, b, wait_prev_scatter=True)
            start_gather(c + 1, (b + 1) % NBUF)
        return 0

    lax.fori_loop(1, NCH // NBUF - 1, outer, 0)

    for c in range(NCH - NBUF, NCH):
        b = c % NBUF
        wait_gather(c, b)
        s_desc(c, b).start()
        if c + 1 < NCH:
            nb = (b + 1) % NBUF
            s_desc(c + 1 - NBUF, nb).wait()
            start_gather(c + 1, nb)
    for c in range(NCH - NBUF, NCH):
        s_desc(c, c % NBUF).wait()


def kernel(grid, color_table, row_table, col_table):
    fused = _build_fused(color_table, row_table, col_table)
    out = _sc_gather(fused, grid.reshape(B * P))
    return out.reshape(B, H, W, D_MODEL)


# speedup vs baseline: 1.5336x; 1.0257x over previous
import functools

import jax
import jax.numpy as jnp
from jax import lax
from jax.experimental import pallas as pl
from jax.experimental.pallas import tpu as pltpu
from jax.experimental.pallas import tpu_sc as plsc

D_MODEL = 128
H = 30
W = 30
NCOLORS = 11
P = H * W
B = 1024
NPL = B * H
NC, NS = 2, 16
NW = NC * NS
PPW = NPL // NW
BPW = B // NW
PLCH = 4
NCH = PPW // PLCH
CPW = PPW * W
NBUF = 3


def _fused_body(color_ref, row_ref, col_ref, out_ref):
    out_ref[...] = (color_ref[...][:, None, None, :]
                    + row_ref[...][None, :, None, :]
                    + col_ref[...][None, None, :, :])


def _build_fused(color_table, row_table, col_table):
    out = pl.pallas_call(
        _fused_body,
        out_shape=jax.ShapeDtypeStruct((NCOLORS, H, W, D_MODEL), jnp.float32),
    )(color_table, row_table, col_table)
    return out.reshape(NCOLORS * P, D_MODEL)


_mesh = plsc.VectorSubcoreMesh(core_axis_name="c", subcore_axis_name="s",
                               num_cores=NC, num_subcores=NS)


@functools.partial(
    pl.kernel,
    out_type=jax.ShapeDtypeStruct((B, H, W, D_MODEL), jnp.float32),
    mesh=_mesh,
    compiler_params=pltpu.CompilerParams(use_tc_tiling_on_sc=True),
    scratch_types=[
        pltpu.VMEM((CPW + 16,), jnp.int32),
        pltpu.VMEM((NCH, PLCH * 32), jnp.int32),
        pltpu.VMEM((NBUF, PLCH, W, D_MODEL), jnp.float32),
        [pltpu.SemaphoreType.DMA] * NBUF,
        [pltpu.SemaphoreType.DMA] * NBUF,
    ],
)
def _sc_gather(fused_hbm, grid_hbm, out_hbm, grid_v, idx_v, rows_v,
               gsems, ssems):
    wid = lax.axis_index("s") * NC + lax.axis_index("c")
    bbase = wid * BPW

    pltpu.sync_copy(grid_hbm.at[pl.ds(wid * CPW, CPW)],
                    grid_v.at[pl.ds(0, CPW)])

    iota = lax.iota(jnp.int32, 16)

    def idx_body(c, h0):
        for k in range(PLCH):
            hk = h0 + k
            hk = jnp.where(hk >= H, hk - H, hk)
            f = c * (PLCH * W) + k * W
            pb = hk * W + iota
            idx_v[c, pl.ds(k * 32, 16)] = grid_v[pl.ds(f, 16)] * P + pb
            idx_v[c, pl.ds(k * 32 + 16, 16)] = (
                grid_v[pl.ds(f + 16, 16)] * P + pb + 16)
        h1 = h0 + PLCH
        return jnp.where(h1 >= H, h1 - H, h1)

    lax.fori_loop(0, NCH, idx_body, jnp.int32(0))

    def g_descs(c, b):
        return [pltpu.make_async_copy(
                    fused_hbm.at[idx_v.at[c, pl.ds(k * 32, W)]],
                    rows_v.at[b, k], gsems[b])
                for k in range(PLCH)]

    def start_gather(c, b):
        for d in g_descs(c, b):
            d.start()

    def wait_gather(c, b):
        for d in g_descs(c, b):
            d.wait()

    def s_start(b, bloc, h0):
        bg = bbase + bloc

        @pl.when(h0 != H - 2)
        def _():
            pltpu.make_async_copy(
                rows_v.at[b], out_hbm.at[bg, pl.ds(h0, PLCH)],
                ssems[b]).start()

        @pl.when(h0 == H - 2)
        def _():
            pltpu.make_async_copy(
                rows_v.at[b, pl.ds(0, 2)],
                out_hbm.at[bg, pl.ds(H - 2, 2)], ssems[b]).start()
            pltpu.make_async_copy(
                rows_v.at[b, pl.ds(2, 2)],
                out_hbm.at[bg + 1, pl.ds(0, 2)], ssems[b]).start()

    def s_start_static(c, b):
        h0 = (c * PLCH) % H
        assert h0 != H - 2
        pltpu.make_async_copy(
            rows_v.at[b],
            out_hbm.at[bbase + (c * PLCH) // H, pl.ds(h0, PLCH)],
            ssems[b]).start()

    def s_wait(b):
        pltpu.make_async_copy(
            rows_v.at[b], out_hbm.at[0, pl.ds(0, PLCH)], ssems[b]).wait()

    start_gather(0, 0)
    for c in range(NBUF):
        b = c % NBUF
        wait_gather(c, b)
        s_start_static(c, b)
        nb = (b + 1) % NBUF
        if c == NBUF - 1:
            s_wait(nb)
        start_gather(c + 1, nb)

    def outer(t, state):
        bloc, h0 = state
        for b in range(NBUF):
            c = t * NBUF + b
            wait_gather(c, b)
            s_start(b, bloc, h0)
            nb = (b + 1) % NBUF
            s_wait(nb)
            start_gather(c + 1, nb)
            h1 = h0 + PLCH
            wrap = h1 >= H
            h0 = jnp.where(wrap, h1 - H, h1)
            bloc = bloc + wrap.astype(jnp.int32)
        return bloc, h0

    c0 = NBUF
    lax.fori_loop(1, NCH // NBUF - 1, outer,
                  (jnp.int32((c0 * PLCH) // H), jnp.int32((c0 * PLCH) % H)))

    for c in range(NCH - NBUF, NCH):
        b = c % NBUF
        wait_gather(c, b)
        s_start_static(c, b)
        if c + 1 < NCH:
            nb = (b + 1) % NBUF
            s_wait(nb)
            start_gather(c + 1, nb)
    for c in range(NCH - NBUF, NCH):
        s_wait(c % NBUF)


def kernel(grid, color_table, row_table, col_table):
    fused = _build_fused(color_table, row_table, col_table)
    return _sc_gather(fused, grid.reshape(B * P))
